# baseline (speedup 1.0000x reference)
import functools

import jax
import jax.numpy as jnp
from jax import lax
from jax.experimental import pallas as pl
from jax.experimental.pallas import tpu as pltpu
from jax.experimental.pallas import tpu_sc as plsc

NN = 10000
NE = 320000
NPAIR = 65536
NPAD = 10240
NC, NS = 2, 16
NW = NC * NS
ECH = 80
NCHUNK = NE // ECH
CH_ITERS = -(-NCHUNK // NW)
BN_EPS = 1e-5

def _mesh():
    return plsc.VectorSubcoreMesh(core_axis_name="c", subcore_axis_name="s")


@functools.cache
def _deg_kernel_fn():
    NBUF = 4
    return pl.kernel(
        _deg_body,
        out_type=jax.ShapeDtypeStruct((NC, NPAD), jnp.float32),
        mesh=_mesh(),
        scratch_types=[
            pltpu.VMEM((4, 1, ECH), jnp.int32),
            pltpu.VMEM((ECH,), jnp.float32),
            pltpu.MemorySpace.VMEM_SHARED((NPAD,), jnp.float32),
        ] + [pltpu.SemaphoreType.DMA] * NBUF,
    )


def _deg_body(ei1, zeros1, out, dstbuf, ones, acc, *sems):
    cid = lax.axis_index("c")
    sid = lax.axis_index("s")
    wid = sid * NC + cid
    for i in range(ECH // 16):
        ones[pl.ds(i * 16, 16)] = jnp.ones((16,), jnp.float32)
    span = NPAD // NS
    pltpu.sync_copy(zeros1.at[pl.ds(sid * span, span)],
                    acc.at[pl.ds(sid * span, span)])
    plsc.subcore_barrier()

    def valid(j):
        return (j < CH_ITERS) & ((j * NW + wid) < NCHUNK)

    def fire(j, b):
        @pl.when(valid(j))
        def _():
            c = j * NW + wid
            pltpu.async_copy(ei1.at[pl.ds(c * ECH, ECH)], dstbuf.at[b, 0],
                             sems[b])

    def drain(j, b):
        @pl.when(valid(j))
        def _():
            c = j * NW + wid
            pltpu.make_async_copy(ei1.at[pl.ds(c * ECH, ECH)],
                                  dstbuf.at[b, 0], sems[b]).wait()
            pltpu.sync_copy(ones, acc.at[dstbuf.at[b, 0]], add=True)

    NBUF = 4
    for b in range(NBUF):
        fire(b, b)

    def outer(g, carry):
        for b in range(NBUF):
            j = g * NBUF + b
            drain(j, b)
            fire(j + NBUF, b)
        return carry

    lax.fori_loop(0, -(-CH_ITERS // NBUF), outer, 0)
    plsc.subcore_barrier()
    pltpu.sync_copy(acc.at[pl.ds(sid * span, span)],
                    out.at[cid, pl.ds(sid * span, span)])


@functools.cache
def _make_scatter(D, tc_tiling=True):
    K = 4

    def _scat(ei0, ei1, hs, zeros, out, idxbuf, rows, acc, *sems):
        cid = lax.axis_index("c")
        sid = lax.axis_index("s")
        wid = sid * NC + cid
        rspan = NPAD // NS
        si = sems[:K]
        sg = sems[K:2 * K]
        ss = sems[2 * K:]

        pltpu.sync_copy(zeros.at[pl.ds(sid * rspan, rspan), :],
                        acc.at[pl.ds(sid * rspan, rspan), :])
        plsc.subcore_barrier()

        def valid(j):
            return (j >= 0) & (j < CH_ITERS) & ((j * NW + wid) < NCHUNK)

        def visit(v, b):
            @pl.when(valid(v - K))
            def _():
                pltpu.make_async_copy(rows.at[b], acc.at[idxbuf.at[b, 1]],
                                      ss[b]).wait()

            @pl.when(valid(v))
            def _():
                c = v * NW + wid
                pltpu.async_copy(ei0.at[pl.ds(c * ECH, ECH)],
                                 idxbuf.at[b, 0], si[b])
                pltpu.async_copy(ei1.at[pl.ds(c * ECH, ECH)],
                                 idxbuf.at[b, 1], si[b])

            bg = (b - 1) % K

            @pl.when(valid(v - 1))
            def _():
                c = (v - 1) * NW + wid
                pltpu.make_async_copy(ei0.at[pl.ds(c * ECH, ECH)],
                                      idxbuf.at[bg, 0], si[bg]).wait()
                pltpu.make_async_copy(ei1.at[pl.ds(c * ECH, ECH)],
                                      idxbuf.at[bg, 1], si[bg]).wait()
                pltpu.async_copy(hs.at[idxbuf.at[bg, 0]], rows.at[bg], sg[bg])

            bs = (b - 3) % K

            @pl.when(valid(v - 3))
            def _():
                pltpu.make_async_copy(hs.at[idxbuf.at[bs, 0]], rows.at[bs],
                                      sg[bs]).wait()
                pltpu.async_copy(rows.at[bs], acc.at[idxbuf.at[bs, 1]],
                                 ss[bs], add=True)

        def outer(g, carry):
            for b in range(K):
                visit(g * K + b, b)
            return carry

        lax.fori_loop(0, -(-(CH_ITERS + K) // K), outer, 0)
        plsc.subcore_barrier()
        pltpu.sync_copy(acc.at[pl.ds(sid * rspan, rspan), :],
                        out.at[cid, pl.ds(sid * rspan, rspan), :])

    return pl.kernel(
        _scat,
        out_type=jax.ShapeDtypeStruct((NC, NPAD, D), jnp.float32),
        mesh=_mesh(),
        compiler_params=pltpu.CompilerParams(use_tc_tiling_on_sc=tc_tiling),
        scratch_types=[
            pltpu.VMEM((K, 2, ECH), jnp.int32),
            pltpu.VMEM((K, ECH, D), jnp.float32),
            pltpu.MemorySpace.VMEM_SHARED((NPAD, D), jnp.float32),
        ] + [pltpu.SemaphoreType.DMA] * (3 * K),
    )


PCH = 128
PPW = NPAIR // NW


PBUF = 2
P_ITERS = PPW // PCH


@functools.cache
def _pairs_kernel_fn():
    return pl.kernel(
        _pairs_body,
        out_type=(jax.ShapeDtypeStruct((NPAIR, 64), jnp.float32),
                  jax.ShapeDtypeStruct((NPAIR, 64), jnp.float32)),
        mesh=_mesh(),
        compiler_params=pltpu.CompilerParams(use_tc_tiling_on_sc=False),
        scratch_types=[
            pltpu.VMEM((PBUF, 2, PCH), jnp.int32),
            pltpu.VMEM((PBUF, PCH, 64), jnp.float32),
            pltpu.VMEM((PBUF, PCH, 64), jnp.float32),
        ] + [pltpu.SemaphoreType.DMA] * (2 * PBUF),
    )


def _pairs_body(a, b, src, dst, ga_out, gb_out, idxbuf, bufa, bufb, *sems):
    cid = lax.axis_index("c")
    sid = lax.axis_index("s")
    wid = sid * NC + cid
    sa = sems[:PBUF]
    sb = sems[PBUF:2 * PBUF]

    def fire(j, bslot):
        @pl.when(j < P_ITERS)
        def _():
            base = wid * PPW + j * PCH
            pltpu.sync_copy(src.at[pl.ds(base, PCH)], idxbuf.at[bslot, 0])
            pltpu.sync_copy(dst.at[pl.ds(base, PCH)], idxbuf.at[bslot, 1])
            pltpu.async_copy(a.at[idxbuf.at[bslot, 0]], bufa.at[bslot],
                             sa[bslot])
            pltpu.async_copy(b.at[idxbuf.at[bslot, 1]], bufb.at[bslot],
                             sb[bslot])

    def drain(j, bslot):
        @pl.when(j < P_ITERS)
        def _():
            base = wid * PPW + j * PCH
            pltpu.make_async_copy(a.at[idxbuf.at[bslot, 0]], bufa.at[bslot],
                                  sa[bslot]).wait()
            pltpu.make_async_copy(b.at[idxbuf.at[bslot, 1]], bufb.at[bslot],
                                  sb[bslot]).wait()
            pltpu.sync_copy(bufa.at[bslot], ga_out.at[pl.ds(base, PCH), :])
            pltpu.sync_copy(bufb.at[bslot], gb_out.at[pl.ds(base, PCH), :])

    for bslot in range(PBUF):
        fire(bslot, bslot)

    def outer(g, carry):
        for bslot in range(PBUF):
            j = g * PBUF + bslot
            drain(j, bslot)
            fire(j + PBUF, bslot)
        return carry

    lax.fori_loop(0, -(-P_ITERS // PBUF), outer, 0)


BLK = 2048


def _mm1_body(deg_ref, x_ref, w1_ref, g1_ref, hs_ref, dinv_ref):
    deg = deg_ref[0, :] + deg_ref[1, :] + 1.0
    dinv = lax.rsqrt(deg)
    a1 = g1_ref[...] * lax.rsqrt(jnp.float32(1.0 + BN_EPS))
    wf = w1_ref[...] * a1[None, :]
    hs_ref[...] = jnp.dot(x_ref[...] * dinv[:, None], wf,
                          preferred_element_type=jnp.float32)
    dinv_ref[...] = dinv


def _mm1(deg, x, W1, g1):
    grid = -(-NN // BLK)
    return pl.pallas_call(
        _mm1_body,
        grid=(grid,),
        in_specs=[
            pl.BlockSpec((NC, BLK), lambda i: (0, i)),
            pl.BlockSpec((BLK, 128), lambda i: (i, 0)),
            pl.BlockSpec((128, 128), lambda i: (0, 0)),
            pl.BlockSpec((128,), lambda i: (0,)),
        ],
        out_specs=[
            pl.BlockSpec((BLK, 128), lambda i: (i, 0)),
            pl.BlockSpec((BLK,), lambda i: (i,)),
        ],
        out_shape=[
            jax.ShapeDtypeStruct((NN, 128), jnp.float32),
            jax.ShapeDtypeStruct((NN,), jnp.float32),
        ],
    )(deg, x, W1, g1)


def _mm2_body(s_ref, hs1_ref, dinv_ref, w2_ref, g1_ref, b1_ref, be1_ref,
              g2_ref, out_ref):
    inv_s = lax.rsqrt(jnp.float32(1.0 + BN_EPS))
    a1 = g1_ref[...] * inv_s
    bf1 = a1 * b1_ref[...] + be1_ref[...]
    s = s_ref[0] + s_ref[1] + hs1_ref[...]
    dinv = dinv_ref[...]
    z1 = jnp.maximum(dinv[:, None] * s + bf1[None, :], 0.0)
    a2 = g2_ref[...] * inv_s
    wf = w2_ref[...] * a2[None, :]
    out_ref[...] = jnp.dot(z1 * dinv[:, None], wf,
                           preferred_element_type=jnp.float32)


def _mm2(s1, hs1, dinv, W2, g1, b1, be1, g2):
    grid = -(-NN // BLK)
    return pl.pallas_call(
        _mm2_body,
        grid=(grid,),
        in_specs=[
            pl.BlockSpec((NC, BLK, 128), lambda i: (0, i, 0)),
            pl.BlockSpec((BLK, 128), lambda i: (i, 0)),
            pl.BlockSpec((BLK,), lambda i: (i,)),
            pl.BlockSpec((128, 64), lambda i: (0, 0)),
            pl.BlockSpec((128,), lambda i: (0,)),
            pl.BlockSpec((128,), lambda i: (0,)),
            pl.BlockSpec((128,), lambda i: (0,)),
            pl.BlockSpec((64,), lambda i: (0,)),
        ],
        out_specs=pl.BlockSpec((BLK, 64), lambda i: (i, 0)),
        out_shape=jax.ShapeDtypeStruct((NN, 64), jnp.float32),
    )(s1, hs1, dinv, W2, g1, b1, be1, g2)


def _mm3_body(s_ref, hs2_ref, dinv_ref, wh1_ref, bh1_ref, g2_ref, b2_ref,
              be2_ref, a_ref, b_ref):
    inv_s = lax.rsqrt(jnp.float32(1.0 + BN_EPS))
    a2 = g2_ref[...] * inv_s
    bf2 = a2 * b2_ref[...] + be2_ref[...]
    s = s_ref[0] + s_ref[1] + hs2_ref[...]
    z2 = jnp.maximum(dinv_ref[...][:, None] * s + bf2[None, :], 0.0)
    u = wh1_ref[:64, :]
    v = wh1_ref[64:, :]
    a_ref[...] = (jnp.dot(z2, u, preferred_element_type=jnp.float32)
                  + bh1_ref[...][None, :])
    b_ref[...] = jnp.dot(z2, v, preferred_element_type=jnp.float32)


def _mm3(s2, hs2, dinv, Wh1, bh1, g2, b2, be2):
    grid = -(-NN // BLK)
    return pl.pallas_call(
        _mm3_body,
        grid=(grid,),
        in_specs=[
            pl.BlockSpec((NC, BLK, 64), lambda i: (0, i, 0)),
            pl.BlockSpec((BLK, 64), lambda i: (i, 0)),
            pl.BlockSpec((BLK,), lambda i: (i,)),
            pl.BlockSpec((128, 64), lambda i: (0, 0)),
            pl.BlockSpec((64,), lambda i: (0,)),
            pl.BlockSpec((64,), lambda i: (0,)),
            pl.BlockSpec((64,), lambda i: (0,)),
            pl.BlockSpec((64,), lambda i: (0,)),
        ],
        out_specs=[
            pl.BlockSpec((BLK, 64), lambda i: (i, 0)),
            pl.BlockSpec((BLK, 64), lambda i: (i, 0)),
        ],
        out_shape=[
            jax.ShapeDtypeStruct((NN, 64), jnp.float32),
            jax.ShapeDtypeStruct((NN, 64), jnp.float32),
        ],
    )(s2, hs2, dinv, Wh1, bh1, g2, b2, be2)


TBLK = 8192


def _tail_body(ga_ref, gb_ref, wh2_ref, bh2_ref, out_ref):
    h = jnp.maximum(ga_ref[...] + gb_ref[...], 0.0)
    logits = jnp.dot(h, wh2_ref[...], preferred_element_type=jnp.float32)
    out_ref[...] = jax.nn.sigmoid(logits + bh2_ref[0])


def _tail(ga, gb, Wh2, bh2):
    grid = NPAIR // TBLK
    return pl.pallas_call(
        _tail_body,
        grid=(grid,),
        in_specs=[
            pl.BlockSpec((TBLK, 64), lambda i: (i, 0)),
            pl.BlockSpec((TBLK, 64), lambda i: (i, 0)),
            pl.BlockSpec((64, 1), lambda i: (0, 0)),
            pl.BlockSpec((1,), lambda i: (0,)),
        ],
        out_specs=pl.BlockSpec((TBLK, 1), lambda i: (i, 0)),
        out_shape=jax.ShapeDtypeStruct((NPAIR, 1), jnp.float32),
    )(ga, gb, Wh2, bh2)


def kernel(x, ei, src, dst, W1, b1, g1, be1, W2, b2, g2, be2, Wh1, bh1, Wh2,
           bh2):
    zeros1 = jnp.zeros((NPAD,), jnp.float32)
    zeros128 = jnp.zeros((NPAD, 128), jnp.float32)
    zeros64 = jnp.zeros((NPAD, 64), jnp.float32)

    ei0 = ei[0]
    ei1 = ei[1]
    deg = _deg_kernel_fn()(ei1, zeros1)
    hs1, dinv = _mm1(deg, x, W1, g1)
    s1 = _make_scatter(128)(ei0, ei1, hs1, zeros128)
    hs2 = _mm2(s1, hs1, dinv, W2, g1, b1, be1, g2)
    s2 = _make_scatter(64, tc_tiling=False)(ei0, ei1, hs2, zeros64)
    a, b = _mm3(s2, hs2, dinv, Wh1, bh1, g2, b2, be2)
    ga, gb = _pairs_kernel_fn()(a, b, src, dst)
    out = _tail(ga, gb, Wh2, bh2)
    return out[:, 0]


# speedup vs baseline: 1.1114x; 1.1114x over previous
import functools

import jax
import jax.numpy as jnp
from jax import lax
from jax.experimental import pallas as pl
from jax.experimental.pallas import tpu as pltpu
from jax.experimental.pallas import tpu_sc as plsc

NN = 10000
NE = 320000
NPAIR = 65536
NPAD = 10240
NC, NS = 2, 16
NW = NC * NS
ECH = 80
NCHUNK = NE // ECH
CH_ITERS = -(-NCHUNK // NW)
BN_EPS = 1e-5

def _mesh():
    return plsc.VectorSubcoreMesh(core_axis_name="c", subcore_axis_name="s")


@functools.cache
def _deg_kernel_fn():
    NBUF = 4
    return pl.kernel(
        _deg_body,
        out_type=jax.ShapeDtypeStruct((NC, NPAD), jnp.float32),
        mesh=_mesh(),
        scratch_types=[
            pltpu.VMEM((4, 1, ECH), jnp.int32),
            pltpu.VMEM((ECH,), jnp.float32),
            pltpu.MemorySpace.VMEM_SHARED((NPAD,), jnp.float32),
        ] + [pltpu.SemaphoreType.DMA] * NBUF,
    )


def _deg_body(ei1, zeros1, out, dstbuf, ones, acc, *sems):
    cid = lax.axis_index("c")
    sid = lax.axis_index("s")
    wid = sid * NC + cid
    for i in range(ECH // 16):
        ones[pl.ds(i * 16, 16)] = jnp.ones((16,), jnp.float32)
    span = NPAD // NS
    pltpu.sync_copy(zeros1.at[pl.ds(sid * span, span)],
                    acc.at[pl.ds(sid * span, span)])
    plsc.subcore_barrier()

    def valid(j):
        return (j < CH_ITERS) & ((j * NW + wid) < NCHUNK)

    def fire(j, b):
        @pl.when(valid(j))
        def _():
            c = j * NW + wid
            pltpu.async_copy(ei1.at[pl.ds(c * ECH, ECH)], dstbuf.at[b, 0],
                             sems[b])

    def drain(j, b):
        @pl.when(valid(j))
        def _():
            c = j * NW + wid
            pltpu.make_async_copy(ei1.at[pl.ds(c * ECH, ECH)],
                                  dstbuf.at[b, 0], sems[b]).wait()
            pltpu.sync_copy(ones, acc.at[dstbuf.at[b, 0]], add=True)

    NBUF = 4
    for b in range(NBUF):
        fire(b, b)

    def outer(g, carry):
        for b in range(NBUF):
            j = g * NBUF + b
            drain(j, b)
            fire(j + NBUF, b)
        return carry

    lax.fori_loop(0, -(-CH_ITERS // NBUF), outer, 0)
    plsc.subcore_barrier()
    pltpu.sync_copy(acc.at[pl.ds(sid * span, span)],
                    out.at[cid, pl.ds(sid * span, span)])


@functools.cache
def _make_scatter(D, tc_tiling=True):
    K = 4

    def _scat(ei0, ei1, hs, zeros, out, idxbuf, rows, acc, *sems):
        cid = lax.axis_index("c")
        sid = lax.axis_index("s")
        wid = sid * NC + cid
        rspan = NPAD // NS
        si = sems[:K]
        sg = sems[K:2 * K]
        ss = sems[2 * K:]

        pltpu.sync_copy(zeros.at[pl.ds(sid * rspan, rspan), :],
                        acc.at[pl.ds(sid * rspan, rspan), :])
        plsc.subcore_barrier()

        def valid(j):
            return (j >= 0) & (j < CH_ITERS) & ((j * NW + wid) < NCHUNK)

        def visit(v, b):
            @pl.when(valid(v - K))
            def _():
                pltpu.make_async_copy(rows.at[b], acc.at[idxbuf.at[b, 1]],
                                      ss[b]).wait()

            @pl.when(valid(v))
            def _():
                c = v * NW + wid
                pltpu.async_copy(ei0.at[pl.ds(c * ECH, ECH)],
                                 idxbuf.at[b, 0], si[b])
                pltpu.async_copy(ei1.at[pl.ds(c * ECH, ECH)],
                                 idxbuf.at[b, 1], si[b])

            bg = (b - 1) % K

            @pl.when(valid(v - 1))
            def _():
                c = (v - 1) * NW + wid
                pltpu.make_async_copy(ei0.at[pl.ds(c * ECH, ECH)],
                                      idxbuf.at[bg, 0], si[bg]).wait()
                pltpu.make_async_copy(ei1.at[pl.ds(c * ECH, ECH)],
                                      idxbuf.at[bg, 1], si[bg]).wait()
                pltpu.async_copy(hs.at[idxbuf.at[bg, 0]], rows.at[bg], sg[bg])

            bs = (b - 3) % K

            @pl.when(valid(v - 3))
            def _():
                pltpu.make_async_copy(hs.at[idxbuf.at[bs, 0]], rows.at[bs],
                                      sg[bs]).wait()
                pltpu.async_copy(rows.at[bs], acc.at[idxbuf.at[bs, 1]],
                                 ss[bs], add=True)

        def outer(g, carry):
            for b in range(K):
                visit(g * K + b, b)
            return carry

        lax.fori_loop(0, -(-(CH_ITERS + K) // K), outer, 0)
        plsc.subcore_barrier()
        pltpu.sync_copy(acc.at[pl.ds(sid * rspan, rspan), :],
                        out.at[cid, pl.ds(sid * rspan, rspan), :])

    return pl.kernel(
        _scat,
        out_type=jax.ShapeDtypeStruct((NC, NPAD, D), jnp.float32),
        mesh=_mesh(),
        compiler_params=pltpu.CompilerParams(use_tc_tiling_on_sc=tc_tiling),
        scratch_types=[
            pltpu.VMEM((K, 2, ECH), jnp.int32),
            pltpu.VMEM((K, ECH, D), jnp.float32),
            pltpu.MemorySpace.VMEM_SHARED((NPAD, D), jnp.float32),
        ] + [pltpu.SemaphoreType.DMA] * (3 * K),
    )


PCH = 128
PPW = NPAIR // NW


PBUF = 2
P_ITERS = PPW // PCH


@functools.cache
def _pairs_kernel_fn():
    return pl.kernel(
        _pairs_body,
        out_type=(jax.ShapeDtypeStruct((NPAIR, 128), jnp.float32),
                  jax.ShapeDtypeStruct((NPAIR, 128), jnp.float32)),
        mesh=_mesh(),
        scratch_types=[
            pltpu.VMEM((PBUF, 2, PCH), jnp.int32),
            pltpu.VMEM((PBUF, PCH, 128), jnp.float32),
            pltpu.VMEM((PBUF, PCH, 128), jnp.float32),
        ] + [pltpu.SemaphoreType.DMA] * (2 * PBUF),
    )


def _pairs_body(ab, src, dst, ga_out, gb_out, idxbuf, bufa, bufb, *sems):
    cid = lax.axis_index("c")
    sid = lax.axis_index("s")
    wid = sid * NC + cid
    sa = sems[:PBUF]
    sb = sems[PBUF:2 * PBUF]

    def fire(j, b):
        @pl.when(j < P_ITERS)
        def _():
            base = wid * PPW + j * PCH
            pltpu.sync_copy(src.at[pl.ds(base, PCH)], idxbuf.at[b, 0])
            pltpu.sync_copy(dst.at[pl.ds(base, PCH)], idxbuf.at[b, 1])
            pltpu.async_copy(ab.at[idxbuf.at[b, 0]], bufa.at[b], sa[b])
            pltpu.async_copy(ab.at[idxbuf.at[b, 1]], bufb.at[b], sb[b])

    def drain(j, b):
        @pl.when(j < P_ITERS)
        def _():
            base = wid * PPW + j * PCH
            pltpu.make_async_copy(ab.at[idxbuf.at[b, 0]], bufa.at[b],
                                  sa[b]).wait()
            pltpu.make_async_copy(ab.at[idxbuf.at[b, 1]], bufb.at[b],
                                  sb[b]).wait()
            pltpu.sync_copy(bufa.at[b], ga_out.at[pl.ds(base, PCH), :])
            pltpu.sync_copy(bufb.at[b], gb_out.at[pl.ds(base, PCH), :])

    for b in range(PBUF):
        fire(b, b)

    def outer(g, carry):
        for b in range(PBUF):
            j = g * PBUF + b
            drain(j, b)
            fire(j + PBUF, b)
        return carry

    lax.fori_loop(0, -(-P_ITERS // PBUF), outer, 0)


BLK = 2048


def _mm1_body(deg_ref, x_ref, w1_ref, g1_ref, hs_ref, dinv_ref):
    deg = deg_ref[0, :] + deg_ref[1, :] + 1.0
    dinv = lax.rsqrt(deg)
    a1 = g1_ref[...] * lax.rsqrt(jnp.float32(1.0 + BN_EPS))
    wf = w1_ref[...] * a1[None, :]
    hs_ref[...] = jnp.dot(x_ref[...] * dinv[:, None], wf,
                          preferred_element_type=jnp.float32)
    dinv_ref[...] = dinv


def _mm1(deg, x, W1, g1):
    grid = -(-NN // BLK)
    return pl.pallas_call(
        _mm1_body,
        grid=(grid,),
        in_specs=[
            pl.BlockSpec((NC, BLK), lambda i: (0, i)),
            pl.BlockSpec((BLK, 128), lambda i: (i, 0)),
            pl.BlockSpec((128, 128), lambda i: (0, 0)),
            pl.BlockSpec((128,), lambda i: (0,)),
        ],
        out_specs=[
            pl.BlockSpec((BLK, 128), lambda i: (i, 0)),
            pl.BlockSpec((BLK,), lambda i: (i,)),
        ],
        out_shape=[
            jax.ShapeDtypeStruct((NN, 128), jnp.float32),
            jax.ShapeDtypeStruct((NN,), jnp.float32),
        ],
    )(deg, x, W1, g1)


def _mm2_body(s_ref, hs1_ref, dinv_ref, w2_ref, g1_ref, b1_ref, be1_ref,
              g2_ref, out_ref):
    inv_s = lax.rsqrt(jnp.float32(1.0 + BN_EPS))
    a1 = g1_ref[...] * inv_s
    bf1 = a1 * b1_ref[...] + be1_ref[...]
    s = s_ref[0] + s_ref[1] + hs1_ref[...]
    dinv = dinv_ref[...]
    z1 = jnp.maximum(dinv[:, None] * s + bf1[None, :], 0.0)
    a2 = g2_ref[...] * inv_s
    wf = w2_ref[...] * a2[None, :]
    out_ref[...] = jnp.dot(z1 * dinv[:, None], wf,
                           preferred_element_type=jnp.float32)


def _mm2(s1, hs1, dinv, W2, g1, b1, be1, g2):
    grid = -(-NN // BLK)
    return pl.pallas_call(
        _mm2_body,
        grid=(grid,),
        in_specs=[
            pl.BlockSpec((NC, BLK, 128), lambda i: (0, i, 0)),
            pl.BlockSpec((BLK, 128), lambda i: (i, 0)),
            pl.BlockSpec((BLK,), lambda i: (i,)),
            pl.BlockSpec((128, 64), lambda i: (0, 0)),
            pl.BlockSpec((128,), lambda i: (0,)),
            pl.BlockSpec((128,), lambda i: (0,)),
            pl.BlockSpec((128,), lambda i: (0,)),
            pl.BlockSpec((64,), lambda i: (0,)),
        ],
        out_specs=pl.BlockSpec((BLK, 64), lambda i: (i, 0)),
        out_shape=jax.ShapeDtypeStruct((NN, 64), jnp.float32),
    )(s1, hs1, dinv, W2, g1, b1, be1, g2)


def _mm3_body(s_ref, hs2_ref, dinv_ref, wh1_ref, bh1_ref, g2_ref, b2_ref,
              be2_ref, ab_ref):
    inv_s = lax.rsqrt(jnp.float32(1.0 + BN_EPS))
    a2 = g2_ref[...] * inv_s
    bf2 = a2 * b2_ref[...] + be2_ref[...]
    s = s_ref[0] + s_ref[1] + hs2_ref[...]
    z2 = jnp.maximum(dinv_ref[...][:, None] * s + bf2[None, :], 0.0)
    u = wh1_ref[:64, :]
    v = wh1_ref[64:, :]
    a = (jnp.dot(z2, u, preferred_element_type=jnp.float32)
         + bh1_ref[...][None, :])
    b = jnp.dot(z2, v, preferred_element_type=jnp.float32)
    ab_ref[...] = jnp.concatenate([a, b], axis=1)


def _mm3(s2, hs2, dinv, Wh1, bh1, g2, b2, be2):
    grid = -(-NN // BLK)
    return pl.pallas_call(
        _mm3_body,
        grid=(grid,),
        in_specs=[
            pl.BlockSpec((NC, BLK, 64), lambda i: (0, i, 0)),
            pl.BlockSpec((BLK, 64), lambda i: (i, 0)),
            pl.BlockSpec((BLK,), lambda i: (i,)),
            pl.BlockSpec((128, 64), lambda i: (0, 0)),
            pl.BlockSpec((64,), lambda i: (0,)),
            pl.BlockSpec((64,), lambda i: (0,)),
            pl.BlockSpec((64,), lambda i: (0,)),
            pl.BlockSpec((64,), lambda i: (0,)),
        ],
        out_specs=pl.BlockSpec((BLK, 128), lambda i: (i, 0)),
        out_shape=jax.ShapeDtypeStruct((NN, 128), jnp.float32),
    )(s2, hs2, dinv, Wh1, bh1, g2, b2, be2)


TBLK = 8192


def _tail_body(ga_ref, gb_ref, wh2_ref, bh2_ref, out_ref):
    h = jnp.maximum(ga_ref[:, :64] + gb_ref[:, 64:], 0.0)
    logits = jnp.dot(h, wh2_ref[...], preferred_element_type=jnp.float32)
    out_ref[...] = jax.nn.sigmoid(logits + bh2_ref[0])


def _tail(ga, gb, Wh2, bh2):
    grid = NPAIR // TBLK
    return pl.pallas_call(
        _tail_body,
        grid=(grid,),
        in_specs=[
            pl.BlockSpec((TBLK, 128), lambda i: (i, 0)),
            pl.BlockSpec((TBLK, 128), lambda i: (i, 0)),
            pl.BlockSpec((64, 1), lambda i: (0, 0)),
            pl.BlockSpec((1,), lambda i: (0,)),
        ],
        out_specs=pl.BlockSpec((TBLK, 1), lambda i: (i, 0)),
        out_shape=jax.ShapeDtypeStruct((NPAIR, 1), jnp.float32),
    )(ga, gb, Wh2, bh2)


def kernel(x, ei, src, dst, W1, b1, g1, be1, W2, b2, g2, be2, Wh1, bh1, Wh2,
           bh2):
    zeros1 = jnp.zeros((NPAD,), jnp.float32)
    zeros128 = jnp.zeros((NPAD, 128), jnp.float32)
    zeros64 = jnp.zeros((NPAD, 64), jnp.float32)

    ei0 = ei[0]
    ei1 = ei[1]
    deg = _deg_kernel_fn()(ei1, zeros1)
    hs1, dinv = _mm1(deg, x, W1, g1)
    s1 = _make_scatter(128)(ei0, ei1, hs1, zeros128)
    hs2 = _mm2(s1, hs1, dinv, W2, g1, b1, be1, g2)
    s2 = _make_scatter(64, tc_tiling=False)(ei0, ei1, hs2, zeros64)
    ab = _mm3(s2, hs2, dinv, Wh1, bh1, g2, b2, be2)
    ga, gb = _pairs_kernel_fn()(ab, src, dst)
    out = _tail(ga, gb, Wh2, bh2)
    return out[:, 0]


# speedup vs baseline: 1.1615x; 1.0451x over previous
import functools

import jax
import jax.numpy as jnp
from jax import lax
from jax.experimental import pallas as pl
from jax.experimental.pallas import tpu as pltpu
from jax.experimental.pallas import tpu_sc as plsc

NN = 10000
NE = 320000
NPAIR = 65536
NPAD = 10240
NC, NS = 2, 16
NW = NC * NS
ECH = 80
NCHUNK = NE // ECH
CH_ITERS = -(-NCHUNK // NW)
BN_EPS = 1e-5

def _mesh():
    return plsc.VectorSubcoreMesh(core_axis_name="c", subcore_axis_name="s")


@functools.cache
def _deg_kernel_fn():
    NBUF = 4
    return pl.kernel(
        _deg_body,
        out_type=jax.ShapeDtypeStruct((NC, NPAD), jnp.float32),
        mesh=_mesh(),
        scratch_types=[
            pltpu.VMEM((4, 1, ECH), jnp.int32),
            pltpu.VMEM((ECH,), jnp.float32),
            pltpu.MemorySpace.VMEM_SHARED((NPAD,), jnp.float32),
        ] + [pltpu.SemaphoreType.DMA] * NBUF,
    )


def _deg_body(ei1, zeros1, out, dstbuf, ones, acc, *sems):
    cid = lax.axis_index("c")
    sid = lax.axis_index("s")
    wid = sid * NC + cid
    for i in range(ECH // 16):
        ones[pl.ds(i * 16, 16)] = jnp.ones((16,), jnp.float32)
    span = NPAD // NS
    pltpu.sync_copy(zeros1.at[pl.ds(sid * span, span)],
                    acc.at[pl.ds(sid * span, span)])
    plsc.subcore_barrier()

    def valid(j):
        return (j < CH_ITERS) & ((j * NW + wid) < NCHUNK)

    def fire(j, b):
        @pl.when(valid(j))
        def _():
            c = j * NW + wid
            pltpu.async_copy(ei1.at[pl.ds(c * ECH, ECH)], dstbuf.at[b, 0],
                             sems[b])

    def drain(j, b):
        @pl.when(valid(j))
        def _():
            c = j * NW + wid
            pltpu.make_async_copy(ei1.at[pl.ds(c * ECH, ECH)],
                                  dstbuf.at[b, 0], sems[b]).wait()
            pltpu.sync_copy(ones, acc.at[dstbuf.at[b, 0]], add=True)

    NBUF = 4
    for b in range(NBUF):
        fire(b, b)

    def outer(g, carry):
        for b in range(NBUF):
            j = g * NBUF + b
            drain(j, b)
            fire(j + NBUF, b)
        return carry

    lax.fori_loop(0, -(-CH_ITERS // NBUF), outer, 0)
    plsc.subcore_barrier()
    pltpu.sync_copy(acc.at[pl.ds(sid * span, span)],
                    out.at[cid, pl.ds(sid * span, span)])


@functools.cache
def _make_scatter(D, tc_tiling=True):
    K = 4

    def _scat(ei0, ei1, hs, zeros, out, idxbuf, rows, acc, *sems):
        cid = lax.axis_index("c")
        sid = lax.axis_index("s")
        wid = sid * NC + cid
        rspan = NPAD // NS
        si = sems[:K]
        sg = sems[K:2 * K]
        ss = sems[2 * K:]

        pltpu.sync_copy(zeros.at[pl.ds(sid * rspan, rspan), :],
                        acc.at[pl.ds(sid * rspan, rspan), :])
        plsc.subcore_barrier()

        def valid(j):
            return (j >= 0) & (j < CH_ITERS) & ((j * NW + wid) < NCHUNK)

        def visit(v, b):
            @pl.when(valid(v - K))
            def _():
                pltpu.make_async_copy(rows.at[b], acc.at[idxbuf.at[b, 1]],
                                      ss[b]).wait()

            @pl.when(valid(v))
            def _():
                c = v * NW + wid
                pltpu.async_copy(ei0.at[pl.ds(c * ECH, ECH)],
                                 idxbuf.at[b, 0], si[b])
                pltpu.async_copy(ei1.at[pl.ds(c * ECH, ECH)],
                                 idxbuf.at[b, 1], si[b])

            bg = (b - 1) % K

            @pl.when(valid(v - 1))
            def _():
                c = (v - 1) * NW + wid
                pltpu.make_async_copy(ei0.at[pl.ds(c * ECH, ECH)],
                                      idxbuf.at[bg, 0], si[bg]).wait()
                pltpu.make_async_copy(ei1.at[pl.ds(c * ECH, ECH)],
                                      idxbuf.at[bg, 1], si[bg]).wait()
                pltpu.async_copy(hs.at[idxbuf.at[bg, 0]], rows.at[bg], sg[bg])

            bs = (b - 3) % K

            @pl.when(valid(v - 3))
            def _():
                pltpu.make_async_copy(hs.at[idxbuf.at[bs, 0]], rows.at[bs],
                                      sg[bs]).wait()
                pltpu.async_copy(rows.at[bs], acc.at[idxbuf.at[bs, 1]],
                                 ss[bs], add=True)

        def outer(g, carry):
            for b in range(K):
                visit(g * K + b, b)
            return carry

        lax.fori_loop(0, -(-(CH_ITERS + K) // K), outer, 0)
        plsc.subcore_barrier()
        pltpu.sync_copy(acc.at[pl.ds(sid * rspan, rspan), :],
                        out.at[cid, pl.ds(sid * rspan, rspan), :])

    return pl.kernel(
        _scat,
        out_type=jax.ShapeDtypeStruct((NC, NPAD, D), jnp.float32),
        mesh=_mesh(),
        compiler_params=pltpu.CompilerParams(use_tc_tiling_on_sc=tc_tiling),
        scratch_types=[
            pltpu.VMEM((K, 2, ECH), jnp.int32),
            pltpu.VMEM((K, ECH, D), jnp.float32),
            pltpu.MemorySpace.VMEM_SHARED((NPAD, D), jnp.float32),
        ] + [pltpu.SemaphoreType.DMA] * (3 * K),
    )


PCH = 128
PPW = NPAIR // NW


PBUF = 2
P_ITERS = PPW // PCH


@functools.cache
def _pairs_kernel_fn():
    return pl.kernel(
        _pairs_body,
        out_type=jax.ShapeDtypeStruct((NPAIR, 128), jnp.float32),
        mesh=_mesh(),
        scratch_types=[
            pltpu.VMEM((PBUF, 2, PCH), jnp.int32),
            pltpu.VMEM((PBUF, PCH, 128), jnp.float32),
            pltpu.VMEM((PBUF, PCH, 128), jnp.float32),
        ] + [pltpu.SemaphoreType.DMA] * (2 * PBUF),
    )


def _pairs_body(ab, src, dst, g_out, idxbuf, bufa, bufb, *sems):
    cid = lax.axis_index("c")
    sid = lax.axis_index("s")
    wid = sid * NC + cid
    sa = sems[:PBUF]
    sb = sems[PBUF:2 * PBUF]

    def fire(j, b):
        @pl.when(j < P_ITERS)
        def _():
            base = wid * PPW + j * PCH
            pltpu.sync_copy(src.at[pl.ds(base, PCH)], idxbuf.at[b, 0])
            pltpu.sync_copy(dst.at[pl.ds(base, PCH)], idxbuf.at[b, 1])
            pltpu.async_copy(ab.at[idxbuf.at[b, 0]], bufa.at[b], sa[b])
            pltpu.async_copy(ab.at[idxbuf.at[b, 1]], bufb.at[b], sb[b])

    def drain(j, b):
        @pl.when(j < P_ITERS)
        def _():
            base = wid * PPW + j * PCH
            pltpu.make_async_copy(ab.at[idxbuf.at[b, 0]], bufa.at[b],
                                  sa[b]).wait()
            pltpu.make_async_copy(ab.at[idxbuf.at[b, 1]], bufb.at[b],
                                  sb[b]).wait()

            def pbody(p4, carry):
                for u in range(4):
                    p = p4 * 4 + u
                    for k in range(4):
                        va = bufa.at[b][p, pl.ds(k * 16, 16)]
                        vb = bufb.at[b][p, pl.ds(64 + k * 16, 16)]
                        bufa.at[b][p, pl.ds(k * 16, 16)] = va + vb
                return carry

            lax.fori_loop(0, PCH // 4, pbody, 0)
            pltpu.sync_copy(bufa.at[b], g_out.at[pl.ds(base, PCH), :])

    for b in range(PBUF):
        fire(b, b)

    def outer(g, carry):
        for b in range(PBUF):
            j = g * PBUF + b
            drain(j, b)
            fire(j + PBUF, b)
        return carry

    lax.fori_loop(0, -(-P_ITERS // PBUF), outer, 0)


BLK = 2048


def _mm1_body(deg_ref, x_ref, w1_ref, g1_ref, hs_ref, dinv_ref):
    deg = deg_ref[0, :] + deg_ref[1, :] + 1.0
    dinv = lax.rsqrt(deg)
    a1 = g1_ref[...] * lax.rsqrt(jnp.float32(1.0 + BN_EPS))
    wf = w1_ref[...] * a1[None, :]
    hs_ref[...] = jnp.dot(x_ref[...] * dinv[:, None], wf,
                          preferred_element_type=jnp.float32)
    dinv_ref[...] = dinv


def _mm1(deg, x, W1, g1):
    grid = -(-NN // BLK)
    return pl.pallas_call(
        _mm1_body,
        grid=(grid,),
        in_specs=[
            pl.BlockSpec((NC, BLK), lambda i: (0, i)),
            pl.BlockSpec((BLK, 128), lambda i: (i, 0)),
            pl.BlockSpec((128, 128), lambda i: (0, 0)),
            pl.BlockSpec((128,), lambda i: (0,)),
        ],
        out_specs=[
            pl.BlockSpec((BLK, 128), lambda i: (i, 0)),
            pl.BlockSpec((BLK,), lambda i: (i,)),
        ],
        out_shape=[
            jax.ShapeDtypeStruct((NN, 128), jnp.float32),
            jax.ShapeDtypeStruct((NN,), jnp.float32),
        ],
    )(deg, x, W1, g1)


def _mm2_body(s_ref, hs1_ref, dinv_ref, w2_ref, g1_ref, b1_ref, be1_ref,
              g2_ref, out_ref):
    inv_s = lax.rsqrt(jnp.float32(1.0 + BN_EPS))
    a1 = g1_ref[...] * inv_s
    bf1 = a1 * b1_ref[...] + be1_ref[...]
    s = s_ref[0] + s_ref[1] + hs1_ref[...]
    dinv = dinv_ref[...]
    z1 = jnp.maximum(dinv[:, None] * s + bf1[None, :], 0.0)
    a2 = g2_ref[...] * inv_s
    wf = w2_ref[...] * a2[None, :]
    out_ref[...] = jnp.dot(z1 * dinv[:, None], wf,
                           preferred_element_type=jnp.float32)


def _mm2(s1, hs1, dinv, W2, g1, b1, be1, g2):
    grid = -(-NN // BLK)
    return pl.pallas_call(
        _mm2_body,
        grid=(grid,),
        in_specs=[
            pl.BlockSpec((NC, BLK, 128), lambda i: (0, i, 0)),
            pl.BlockSpec((BLK, 128), lambda i: (i, 0)),
            pl.BlockSpec((BLK,), lambda i: (i,)),
            pl.BlockSpec((128, 64), lambda i: (0, 0)),
            pl.BlockSpec((128,), lambda i: (0,)),
            pl.BlockSpec((128,), lambda i: (0,)),
            pl.BlockSpec((128,), lambda i: (0,)),
            pl.BlockSpec((64,), lambda i: (0,)),
        ],
        out_specs=pl.BlockSpec((BLK, 64), lambda i: (i, 0)),
        out_shape=jax.ShapeDtypeStruct((NN, 64), jnp.float32),
    )(s1, hs1, dinv, W2, g1, b1, be1, g2)


def _mm3_body(s_ref, hs2_ref, dinv_ref, wh1_ref, bh1_ref, g2_ref, b2_ref,
              be2_ref, ab_ref):
    inv_s = lax.rsqrt(jnp.float32(1.0 + BN_EPS))
    a2 = g2_ref[...] * inv_s
    bf2 = a2 * b2_ref[...] + be2_ref[...]
    s = s_ref[0] + s_ref[1] + hs2_ref[...]
    z2 = jnp.maximum(dinv_ref[...][:, None] * s + bf2[None, :], 0.0)
    u = wh1_ref[:64, :]
    v = wh1_ref[64:, :]
    a = (jnp.dot(z2, u, preferred_element_type=jnp.float32)
         + bh1_ref[...][None, :])
    b = jnp.dot(z2, v, preferred_element_type=jnp.float32)
    ab_ref[...] = jnp.concatenate([a, b], axis=1)


def _mm3(s2, hs2, dinv, Wh1, bh1, g2, b2, be2):
    grid = -(-NN // BLK)
    return pl.pallas_call(
        _mm3_body,
        grid=(grid,),
        in_specs=[
            pl.BlockSpec((NC, BLK, 64), lambda i: (0, i, 0)),
            pl.BlockSpec((BLK, 64), lambda i: (i, 0)),
            pl.BlockSpec((BLK,), lambda i: (i,)),
            pl.BlockSpec((128, 64), lambda i: (0, 0)),
            pl.BlockSpec((64,), lambda i: (0,)),
            pl.BlockSpec((64,), lambda i: (0,)),
            pl.BlockSpec((64,), lambda i: (0,)),
            pl.BlockSpec((64,), lambda i: (0,)),
        ],
        out_specs=pl.BlockSpec((BLK, 128), lambda i: (i, 0)),
        out_shape=jax.ShapeDtypeStruct((NN, 128), jnp.float32),
    )(s2, hs2, dinv, Wh1, bh1, g2, b2, be2)


TBLK = 8192


def _tail_body(g_ref, wh2_ref, bh2_ref, out_ref):
    h = jnp.maximum(g_ref[:, :64], 0.0)
    logits = jnp.dot(h, wh2_ref[...], preferred_element_type=jnp.float32)
    out_ref[...] = jax.nn.sigmoid(logits + bh2_ref[0])


def _tail(g, Wh2, bh2):
    grid = NPAIR // TBLK
    return pl.pallas_call(
        _tail_body,
        grid=(grid,),
        in_specs=[
            pl.BlockSpec((TBLK, 128), lambda i: (i, 0)),
            pl.BlockSpec((64, 1), lambda i: (0, 0)),
            pl.BlockSpec((1,), lambda i: (0,)),
        ],
        out_specs=pl.BlockSpec((TBLK, 1), lambda i: (i, 0)),
        out_shape=jax.ShapeDtypeStruct((NPAIR, 1), jnp.float32),
    )(g, Wh2, bh2)


def kernel(x, ei, src, dst, W1, b1, g1, be1, W2, b2, g2, be2, Wh1, bh1, Wh2,
           bh2):
    zeros1 = jnp.zeros((NPAD,), jnp.float32)
    zeros128 = jnp.zeros((NPAD, 128), jnp.float32)
    zeros64 = jnp.zeros((NPAD, 64), jnp.float32)

    ei0 = ei[0]
    ei1 = ei[1]
    deg = _deg_kernel_fn()(ei1, zeros1)
    hs1, dinv = _mm1(deg, x, W1, g1)
    s1 = _make_scatter(128)(ei0, ei1, hs1, zeros128)
    hs2 = _mm2(s1, hs1, dinv, W2, g1, b1, be1, g2)
    s2 = _make_scatter(64, tc_tiling=False)(ei0, ei1, hs2, zeros64)
    ab = _mm3(s2, hs2, dinv, Wh1, bh1, g2, b2, be2)
    g = _pairs_kernel_fn()(ab, src, dst)
    out = _tail(g, Wh2, bh2)
    return out[:, 0]


# speedup vs baseline: 1.1644x; 1.0024x over previous
import functools

import jax
import jax.numpy as jnp
from jax import lax
from jax.experimental import pallas as pl
from jax.experimental.pallas import tpu as pltpu
from jax.experimental.pallas import tpu_sc as plsc

NN = 10000
NE = 320000
NPAIR = 65536
NPAD = 10240
NC, NS = 2, 16
NW = NC * NS
ECH = 80
NCHUNK = NE // ECH
CH_ITERS = -(-NCHUNK // NW)
BN_EPS = 1e-5

def _mesh():
    return plsc.VectorSubcoreMesh(core_axis_name="c", subcore_axis_name="s")


@functools.cache
def _deg_kernel_fn():
    NBUF = 4
    return pl.kernel(
        _deg_body,
        out_type=jax.ShapeDtypeStruct((NC, NPAD), jnp.float32),
        mesh=_mesh(),
        scratch_types=[
            pltpu.VMEM((4, 1, ECH), jnp.int32),
            pltpu.VMEM((ECH,), jnp.float32),
            pltpu.MemorySpace.VMEM_SHARED((NPAD,), jnp.float32),
        ] + [pltpu.SemaphoreType.DMA] * NBUF,
    )


def _deg_body(ei1, zeros1, out, dstbuf, ones, acc, *sems):
    cid = lax.axis_index("c")
    sid = lax.axis_index("s")
    wid = sid * NC + cid
    for i in range(ECH // 16):
        ones[pl.ds(i * 16, 16)] = jnp.ones((16,), jnp.float32)
    span = NPAD // NS
    pltpu.sync_copy(zeros1.at[pl.ds(sid * span, span)],
                    acc.at[pl.ds(sid * span, span)])
    plsc.subcore_barrier()

    def valid(j):
        return (j < CH_ITERS) & ((j * NW + wid) < NCHUNK)

    def fire(j, b):
        @pl.when(valid(j))
        def _():
            c = j * NW + wid
            pltpu.async_copy(ei1.at[pl.ds(c * ECH, ECH)], dstbuf.at[b, 0],
                             sems[b])

    def drain(j, b):
        @pl.when(valid(j))
        def _():
            c = j * NW + wid
            pltpu.make_async_copy(ei1.at[pl.ds(c * ECH, ECH)],
                                  dstbuf.at[b, 0], sems[b]).wait()
            pltpu.sync_copy(ones, acc.at[dstbuf.at[b, 0]], add=True)

    NBUF = 4
    for b in range(NBUF):
        fire(b, b)

    def outer(g, carry):
        for b in range(NBUF):
            j = g * NBUF + b
            drain(j, b)
            fire(j + NBUF, b)
        return carry

    lax.fori_loop(0, -(-CH_ITERS // NBUF), outer, 0)
    plsc.subcore_barrier()
    pltpu.sync_copy(acc.at[pl.ds(sid * span, span)],
                    out.at[cid, pl.ds(sid * span, span)])


@functools.cache
def _make_scatter(D, tc_tiling=True):
    K = 4

    def _scat(ei0, ei1, hs, zeros, out, idxbuf, rows, acc, *sems):
        cid = lax.axis_index("c")
        sid = lax.axis_index("s")
        wid = sid * NC + cid
        rspan = NPAD // NS
        si = sems[:K]
        sg = sems[K:2 * K]
        ss = sems[2 * K:]

        pltpu.sync_copy(zeros.at[pl.ds(sid * rspan, rspan), :],
                        acc.at[pl.ds(sid * rspan, rspan), :])
        plsc.subcore_barrier()

        def valid(j):
            return (j >= 0) & (j < CH_ITERS) & ((j * NW + wid) < NCHUNK)

        def visit(v, b):
            @pl.when(valid(v - K))
            def _():
                pltpu.make_async_copy(rows.at[b], acc.at[idxbuf.at[b, 1]],
                                      ss[b]).wait()

            @pl.when(valid(v))
            def _():
                c = v * NW + wid
                pltpu.async_copy(ei0.at[pl.ds(c * ECH, ECH)],
                                 idxbuf.at[b, 0], si[b])
                pltpu.async_copy(ei1.at[pl.ds(c * ECH, ECH)],
                                 idxbuf.at[b, 1], si[b])

            bg = (b - 1) % K

            @pl.when(valid(v - 1))
            def _():
                c = (v - 1) * NW + wid
                pltpu.make_async_copy(ei0.at[pl.ds(c * ECH, ECH)],
                                      idxbuf.at[bg, 0], si[bg]).wait()
                pltpu.make_async_copy(ei1.at[pl.ds(c * ECH, ECH)],
                                      idxbuf.at[bg, 1], si[bg]).wait()
                pltpu.async_copy(hs.at[idxbuf.at[bg, 0]], rows.at[bg], sg[bg])

            bs = (b - 3) % K

            @pl.when(valid(v - 3))
            def _():
                pltpu.make_async_copy(hs.at[idxbuf.at[bs, 0]], rows.at[bs],
                                      sg[bs]).wait()
                pltpu.async_copy(rows.at[bs], acc.at[idxbuf.at[bs, 1]],
                                 ss[bs], add=True)

        def outer(g, carry):
            for b in range(K):
                visit(g * K + b, b)
            return carry

        lax.fori_loop(0, -(-(CH_ITERS + K) // K), outer, 0)
        plsc.subcore_barrier()
        pltpu.sync_copy(acc.at[pl.ds(sid * rspan, rspan), :],
                        out.at[cid, pl.ds(sid * rspan, rspan), :])

    return pl.kernel(
        _scat,
        out_type=jax.ShapeDtypeStruct((NC, NPAD, D), jnp.float32),
        mesh=_mesh(),
        compiler_params=pltpu.CompilerParams(use_tc_tiling_on_sc=tc_tiling),
        scratch_types=[
            pltpu.VMEM((K, 2, ECH), jnp.int32),
            pltpu.VMEM((K, ECH, D), jnp.float32),
            pltpu.MemorySpace.VMEM_SHARED((NPAD, D), jnp.float32),
        ] + [pltpu.SemaphoreType.DMA] * (3 * K),
    )


PCH = 128
PPW = NPAIR // NW


PBUF = 3
P_ITERS = PPW // PCH


@functools.cache
def _pairs_kernel_fn():
    return pl.kernel(
        _pairs_body,
        out_type=jax.ShapeDtypeStruct((NPAIR, 128), jnp.float32),
        mesh=_mesh(),
        scratch_types=[
            pltpu.VMEM((PBUF, 2, PCH), jnp.int32),
            pltpu.VMEM((PBUF, PCH, 128), jnp.float32),
            pltpu.VMEM((PBUF, PCH, 128), jnp.float32),
        ] + [pltpu.SemaphoreType.DMA] * (2 * PBUF),
    )


def _pairs_body(ab, src, dst, g_out, idxbuf, bufa, bufb, *sems):
    cid = lax.axis_index("c")
    sid = lax.axis_index("s")
    wid = sid * NC + cid
    sa = sems[:PBUF]
    sb = sems[PBUF:2 * PBUF]

    def fire(j, b):
        @pl.when(j < P_ITERS)
        def _():
            base = wid * PPW + j * PCH
            pltpu.sync_copy(src.at[pl.ds(base, PCH)], idxbuf.at[b, 0])
            pltpu.sync_copy(dst.at[pl.ds(base, PCH)], idxbuf.at[b, 1])
            pltpu.async_copy(ab.at[idxbuf.at[b, 0]], bufa.at[b], sa[b])
            pltpu.async_copy(ab.at[idxbuf.at[b, 1]], bufb.at[b], sb[b])

    def drain(j, b):
        @pl.when(j < P_ITERS)
        def _():
            base = wid * PPW + j * PCH
            pltpu.make_async_copy(ab.at[idxbuf.at[b, 0]], bufa.at[b],
                                  sa[b]).wait()
            pltpu.make_async_copy(ab.at[idxbuf.at[b, 1]], bufb.at[b],
                                  sb[b]).wait()

            def pbody(p4, carry):
                for u in range(4):
                    p = p4 * 4 + u
                    for k in range(4):
                        va = bufa.at[b][p, pl.ds(k * 16, 16)]
                        vb = bufb.at[b][p, pl.ds(64 + k * 16, 16)]
                        bufa.at[b][p, pl.ds(k * 16, 16)] = va + vb
                return carry

            lax.fori_loop(0, PCH // 4, pbody, 0)
            pltpu.sync_copy(bufa.at[b], g_out.at[pl.ds(base, PCH), :])

    for b in range(PBUF):
        fire(b, b)

    def outer(g, carry):
        for b in range(PBUF):
            j = g * PBUF + b
            drain(j, b)
            fire(j + PBUF, b)
        return carry

    lax.fori_loop(0, -(-P_ITERS // PBUF), outer, 0)


BLK = 2048


def _mm1_body(deg_ref, x_ref, w1_ref, g1_ref, hs_ref, dinv_ref):
    deg = deg_ref[0, :] + deg_ref[1, :] + 1.0
    dinv = lax.rsqrt(deg)
    a1 = g1_ref[...] * lax.rsqrt(jnp.float32(1.0 + BN_EPS))
    wf = w1_ref[...] * a1[None, :]
    hs_ref[...] = jnp.dot(x_ref[...] * dinv[:, None], wf,
                          preferred_element_type=jnp.float32)
    dinv_ref[...] = dinv


def _mm1(deg, x, W1, g1):
    grid = -(-NN // BLK)
    return pl.pallas_call(
        _mm1_body,
        grid=(grid,),
        in_specs=[
            pl.BlockSpec((NC, BLK), lambda i: (0, i)),
            pl.BlockSpec((BLK, 128), lambda i: (i, 0)),
            pl.BlockSpec((128, 128), lambda i: (0, 0)),
            pl.BlockSpec((128,), lambda i: (0,)),
        ],
        out_specs=[
            pl.BlockSpec((BLK, 128), lambda i: (i, 0)),
            pl.BlockSpec((BLK,), lambda i: (i,)),
        ],
        out_shape=[
            jax.ShapeDtypeStruct((NN, 128), jnp.float32),
            jax.ShapeDtypeStruct((NN,), jnp.float32),
        ],
    )(deg, x, W1, g1)


def _mm2_body(s_ref, hs1_ref, dinv_ref, w2_ref, g1_ref, b1_ref, be1_ref,
              g2_ref, out_ref):
    inv_s = lax.rsqrt(jnp.float32(1.0 + BN_EPS))
    a1 = g1_ref[...] * inv_s
    bf1 = a1 * b1_ref[...] + be1_ref[...]
    s = s_ref[0] + s_ref[1] + hs1_ref[...]
    dinv = dinv_ref[...]
    z1 = jnp.maximum(dinv[:, None] * s + bf1[None, :], 0.0)
    a2 = g2_ref[...] * inv_s
    wf = w2_ref[...] * a2[None, :]
    out_ref[...] = jnp.dot(z1 * dinv[:, None], wf,
                           preferred_element_type=jnp.float32)


def _mm2(s1, hs1, dinv, W2, g1, b1, be1, g2):
    grid = -(-NN // BLK)
    return pl.pallas_call(
        _mm2_body,
        grid=(grid,),
        in_specs=[
            pl.BlockSpec((NC, BLK, 128), lambda i: (0, i, 0)),
            pl.BlockSpec((BLK, 128), lambda i: (i, 0)),
            pl.BlockSpec((BLK,), lambda i: (i,)),
            pl.BlockSpec((128, 64), lambda i: (0, 0)),
            pl.BlockSpec((128,), lambda i: (0,)),
            pl.BlockSpec((128,), lambda i: (0,)),
            pl.BlockSpec((128,), lambda i: (0,)),
            pl.BlockSpec((64,), lambda i: (0,)),
        ],
        out_specs=pl.BlockSpec((BLK, 64), lambda i: (i, 0)),
        out_shape=jax.ShapeDtypeStruct((NN, 64), jnp.float32),
    )(s1, hs1, dinv, W2, g1, b1, be1, g2)


def _mm3_body(s_ref, hs2_ref, dinv_ref, wh1_ref, bh1_ref, g2_ref, b2_ref,
              be2_ref, ab_ref):
    inv_s = lax.rsqrt(jnp.float32(1.0 + BN_EPS))
    a2 = g2_ref[...] * inv_s
    bf2 = a2 * b2_ref[...] + be2_ref[...]
    s = s_ref[0] + s_ref[1] + hs2_ref[...]
    z2 = jnp.maximum(dinv_ref[...][:, None] * s + bf2[None, :], 0.0)
    u = wh1_ref[:64, :]
    v = wh1_ref[64:, :]
    a = (jnp.dot(z2, u, preferred_element_type=jnp.float32)
         + bh1_ref[...][None, :])
    b = jnp.dot(z2, v, preferred_element_type=jnp.float32)
    ab_ref[...] = jnp.concatenate([a, b], axis=1)


def _mm3(s2, hs2, dinv, Wh1, bh1, g2, b2, be2):
    grid = -(-NN // BLK)
    return pl.pallas_call(
        _mm3_body,
        grid=(grid,),
        in_specs=[
            pl.BlockSpec((NC, BLK, 64), lambda i: (0, i, 0)),
            pl.BlockSpec((BLK, 64), lambda i: (i, 0)),
            pl.BlockSpec((BLK,), lambda i: (i,)),
            pl.BlockSpec((128, 64), lambda i: (0, 0)),
            pl.BlockSpec((64,), lambda i: (0,)),
            pl.BlockSpec((64,), lambda i: (0,)),
            pl.BlockSpec((64,), lambda i: (0,)),
            pl.BlockSpec((64,), lambda i: (0,)),
        ],
        out_specs=pl.BlockSpec((BLK, 128), lambda i: (i, 0)),
        out_shape=jax.ShapeDtypeStruct((NN, 128), jnp.float32),
    )(s2, hs2, dinv, Wh1, bh1, g2, b2, be2)


TBLK = 8192


def _tail_body(g_ref, wh2_ref, bh2_ref, out_ref):
    h = jnp.maximum(g_ref[:, :64], 0.0)
    logits = jnp.dot(h, wh2_ref[...], preferred_element_type=jnp.float32)
    out_ref[...] = jax.nn.sigmoid(logits + bh2_ref[0])


def _tail(g, Wh2, bh2):
    grid = NPAIR // TBLK
    return pl.pallas_call(
        _tail_body,
        grid=(grid,),
        in_specs=[
            pl.BlockSpec((TBLK, 128), lambda i: (i, 0)),
            pl.BlockSpec((64, 1), lambda i: (0, 0)),
            pl.BlockSpec((1,), lambda i: (0,)),
        ],
        out_specs=pl.BlockSpec((TBLK, 1), lambda i: (i, 0)),
        out_shape=jax.ShapeDtypeStruct((NPAIR, 1), jnp.float32),
    )(g, Wh2, bh2)


def kernel(x, ei, src, dst, W1, b1, g1, be1, W2, b2, g2, be2, Wh1, bh1, Wh2,
           bh2):
    zeros1 = jnp.zeros((NPAD,), jnp.float32)
    zeros128 = jnp.zeros((NPAD, 128), jnp.float32)
    zeros64 = jnp.zeros((NPAD, 64), jnp.float32)

    ei0 = ei[0]
    ei1 = ei[1]
    deg = _deg_kernel_fn()(ei1, zeros1)
    hs1, dinv = _mm1(deg, x, W1, g1)
    s1 = _make_scatter(128)(ei0, ei1, hs1, zeros128)
    hs2 = _mm2(s1, hs1, dinv, W2, g1, b1, be1, g2)
    s2 = _make_scatter(64, tc_tiling=False)(ei0, ei1, hs2, zeros64)
    ab = _mm3(s2, hs2, dinv, Wh1, bh1, g2, b2, be2)
    g = _pairs_kernel_fn()(ab, src, dst)
    out = _tail(g, Wh2, bh2)
    return out[:, 0]
